# baseline (speedup 1.0000x reference)
import jax
import jax.numpy as jnp
from jax.experimental import pallas as pl
from jax.experimental.pallas import tpu as pltpu

_B = 16384
_FEAT = 128
_CPAD = 16
_BLK = 16384
_GRID = _B // _BLK

_DN_T = (((1,), (1,)), ((), ()))


def _body(x_ref, lab_ref, c_ref, out_ref, acc_ref):
    i = pl.program_id(0)
    x = x_ref[...]
    c = c_ref[...]
    cpad = jnp.concatenate([c, jnp.zeros((_CPAD - 9, _FEAT), jnp.float32)], axis=0)
    lab = lab_ref[0]

    g_t = jax.lax.dot_general(
        cpad, x, _DN_T, preferred_element_type=jnp.float32
    )
    rown_t = jax.lax.dot_general(
        jnp.ones((1, _FEAT), jnp.float32), x * x, _DN_T,
        preferred_element_type=jnp.float32,
    )
    cn2 = jnp.sum(cpad * cpad, axis=1, keepdims=True)

    d2_t = jnp.maximum(rown_t + cn2 - 2.0 * g_t, 0.0)
    dist_t = jnp.sqrt(d2_t)

    classes = jax.lax.broadcasted_iota(jnp.int32, (_CPAD, _BLK), 0)
    onehot_t = (lab == classes).astype(jnp.float32)
    s = jnp.sum(onehot_t * dist_t, axis=1, keepdims=True)
    cnt = jnp.sum(onehot_t, axis=1, keepdims=True)

    part = jnp.concatenate([s, cnt], axis=1)
    prev = jnp.where(i == 0, jnp.zeros_like(part), acc_ref[...])
    acc = prev + part
    acc_ref[...] = acc

    @pl.when(i == pl.num_programs(0) - 1)
    def _():
        s_t = acc[:, 0:1]
        c_t = acc[:, 1:2]
        contrib = jnp.where(c_t > 0.0, s_t / c_t, 0.0)
        out_ref[0, 0] = jnp.sum(contrib) / _B


def kernel(coordinate, labels, center):
    lab3 = labels.reshape(_GRID, 1, _BLK)
    out = pl.pallas_call(
        _body,
        grid=(_GRID,),
        in_specs=[
            pl.BlockSpec((_BLK, _FEAT), lambda i: (i, 0)),
            pl.BlockSpec((1, 1, _BLK), lambda i: (i, 0, 0)),
            pl.BlockSpec((9, _FEAT), lambda i: (0, 0)),
        ],
        out_specs=pl.BlockSpec(memory_space=pltpu.SMEM),
        out_shape=jax.ShapeDtypeStruct((1, 1), jnp.float32),
        scratch_shapes=[pltpu.VMEM((_CPAD, 2), jnp.float32)],
    )(coordinate, lab3, center)
    return out[0, 0]


# speedup vs baseline: 1.1325x; 1.1325x over previous
import jax
import jax.numpy as jnp
from jax.experimental import pallas as pl
from jax.experimental.pallas import tpu as pltpu

_B = 16384
_FEAT = 128
_CPAD = 16
_BLK = 8192
_GRID = _B // _BLK

_DN_T = (((1,), (1,)), ((), ()))


def _body(x_ref, lab_ref, c_ref, out_ref, acc_ref):
    i = pl.program_id(0)
    x = x_ref[...]
    c = c_ref[...]
    cpad = jnp.concatenate([c, jnp.zeros((_CPAD - 9, _FEAT), jnp.float32)], axis=0)
    lab = lab_ref[0]

    x_bf = x.astype(jnp.bfloat16)
    g_t = jax.lax.dot_general(
        cpad.astype(jnp.bfloat16), x_bf, _DN_T,
        preferred_element_type=jnp.float32,
    )
    rown_t = jax.lax.dot_general(
        jnp.ones((1, _FEAT), jnp.bfloat16), x_bf * x_bf, _DN_T,
        preferred_element_type=jnp.float32,
    )
    cn2 = jnp.sum(cpad * cpad, axis=1, keepdims=True)

    d2_t = jnp.maximum(rown_t + cn2 - 2.0 * g_t, 0.0)
    dist_t = jnp.sqrt(d2_t)

    classes = jax.lax.broadcasted_iota(jnp.int32, (_CPAD, _BLK), 0)
    onehot_t = (lab == classes).astype(jnp.float32)
    s = jnp.sum(onehot_t * dist_t, axis=1, keepdims=True)
    cnt = jnp.sum(onehot_t, axis=1, keepdims=True)

    part = jnp.concatenate([s, cnt], axis=1)
    prev = jnp.where(i == 0, jnp.zeros_like(part), acc_ref[...])
    acc = prev + part
    acc_ref[...] = acc

    @pl.when(i == pl.num_programs(0) - 1)
    def _():
        s_t = acc[:, 0:1]
        c_t = acc[:, 1:2]
        contrib = jnp.where(c_t > 0.0, s_t / c_t, 0.0)
        out_ref[0, 0] = jnp.sum(contrib) / _B


def kernel(coordinate, labels, center):
    lab3 = labels.reshape(_GRID, 1, _BLK)
    out = pl.pallas_call(
        _body,
        grid=(_GRID,),
        in_specs=[
            pl.BlockSpec((_BLK, _FEAT), lambda i: (i, 0)),
            pl.BlockSpec((1, 1, _BLK), lambda i: (i, 0, 0)),
            pl.BlockSpec((9, _FEAT), lambda i: (0, 0)),
        ],
        out_specs=pl.BlockSpec(memory_space=pltpu.SMEM),
        out_shape=jax.ShapeDtypeStruct((1, 1), jnp.float32),
        scratch_shapes=[pltpu.VMEM((_CPAD, 2), jnp.float32)],
    )(coordinate, lab3, center)
    return out[0, 0]


# speedup vs baseline: 1.1343x; 1.0015x over previous
import jax
import jax.numpy as jnp
from jax.experimental import pallas as pl
from jax.experimental.pallas import tpu as pltpu

_B = 16384
_FEAT = 128
_CPAD = 16
_BLK = 8192
_GRID = _B // _BLK

_DN_T = (((1,), (1,)), ((), ()))


def _body(x_ref, lab_ref, c_ref, out_ref, acc_ref):
    i = pl.program_id(0)
    x = x_ref[...]
    c = c_ref[...]
    cpad = jnp.concatenate([c, jnp.zeros((_CPAD - 9, _FEAT), jnp.float32)], axis=0)
    lab = lab_ref[0]

    g_t = jax.lax.dot_general(
        cpad, x, _DN_T, preferred_element_type=jnp.float32
    )
    rown_t = jax.lax.dot_general(
        jnp.ones((1, _FEAT), jnp.float32), x * x, _DN_T,
        preferred_element_type=jnp.float32,
    )
    cn2 = jnp.sum(cpad * cpad, axis=1, keepdims=True)

    d2_t = jnp.maximum(rown_t + cn2 - 2.0 * g_t, 0.0)
    dist_t = jnp.sqrt(d2_t)

    classes = jax.lax.broadcasted_iota(jnp.int32, (_CPAD, _BLK), 0)
    onehot_t = (lab == classes).astype(jnp.float32)
    ones_row = jnp.ones((1, _BLK), jnp.float32)
    s = jax.lax.dot_general(
        onehot_t * dist_t, ones_row, _DN_T, preferred_element_type=jnp.float32
    )
    cnt = jax.lax.dot_general(
        onehot_t, ones_row, _DN_T, preferred_element_type=jnp.float32
    )

    part = jnp.concatenate([s, cnt], axis=1)
    prev = jnp.where(i == 0, jnp.zeros_like(part), acc_ref[...])
    acc = prev + part
    acc_ref[...] = acc

    @pl.when(i == pl.num_programs(0) - 1)
    def _():
        s_t = acc[:, 0:1]
        c_t = acc[:, 1:2]
        contrib = jnp.where(c_t > 0.0, s_t / c_t, 0.0)
        out_ref[0, 0] = jnp.sum(contrib) / _B


def kernel(coordinate, labels, center):
    lab3 = labels.reshape(_GRID, 1, _BLK)
    out = pl.pallas_call(
        _body,
        grid=(_GRID,),
        in_specs=[
            pl.BlockSpec((_BLK, _FEAT), lambda i: (i, 0)),
            pl.BlockSpec((1, 1, _BLK), lambda i: (i, 0, 0)),
            pl.BlockSpec((9, _FEAT), lambda i: (0, 0)),
        ],
        out_specs=pl.BlockSpec(memory_space=pltpu.SMEM),
        out_shape=jax.ShapeDtypeStruct((1, 1), jnp.float32),
        scratch_shapes=[pltpu.VMEM((_CPAD, 2), jnp.float32)],
    )(coordinate, lab3, center)
    return out[0, 0]
